# baseline (speedup 1.0000x reference)
import functools

import jax
import jax.numpy as jnp
from jax import lax
from jax.experimental import pallas as pl
from jax.experimental.pallas import tpu as pltpu
from jax.experimental.pallas import tpu_sc as plsc

_L = 16
_H = 1024
_W = 1024
_C = 32
_NW = 32
_N = _H * _W

_mesh = plsc.VectorSubcoreMesh(core_axis_name="c", subcore_axis_name="s")
_params = pltpu.CompilerParams(
    use_tc_tiling_on_sc=False, needs_layout_passes=False
)


@functools.partial(
    pl.kernel,
    out_type=jax.ShapeDtypeStruct((_N, _C), jnp.float32),
    mesh=_mesh,
    scratch_types=[
        pltpu.VMEM((4, 8, 8, 128), jnp.float32),
        pltpu.VMEM((_W, _C), jnp.float32),
    ],
    compiler_params=_params,
)
def _convert(tex5_hbm, table_hbm, in_v, out_v):
    cid = lax.axis_index("c")
    sid = lax.axis_index("s")
    wid = sid * 2 + cid
    planes = _H // _NW

    def plane_body(i, carry):
        y = wid * planes + i
        for ct in range(4):
            pltpu.sync_copy(tex5_hbm.at[y, ct], in_v.at[ct])

        def grp(g, c):
            xt = g // 8
            xg = g - xt * 8
            pvec = g * _L + lax.iota(jnp.int32, _L)
            for ch in range(_C):
                val = in_v[ch // 8, xt, ch % 8, pl.ds(xg * _L, _L)]
                cvec = jnp.full((_L,), ch, jnp.int32)
                plsc.store_scatter(out_v, [pvec, cvec], val)
            return c

        lax.fori_loop(0, _W // _L, grp, 0)
        pltpu.sync_copy(out_v, table_hbm.at[pl.ds(y * _W, _W)])
        return carry

    lax.fori_loop(0, planes, plane_body, 0)


@functools.partial(
    pl.kernel,
    out_type=jax.ShapeDtypeStruct((_H, 4, 8, 8, 128), jnp.float32),
    mesh=_mesh,
    scratch_types=[
        pltpu.VMEM((128,), jnp.float32),
        pltpu.VMEM((128,), jnp.float32),
        pltpu.VMEM((128,), jnp.float32),
        pltpu.VMEM((128,), jnp.float32),
        pltpu.VMEM((128,), jnp.int32),
        pltpu.VMEM((128,), jnp.int32),
        pltpu.VMEM((128,), jnp.int32),
        pltpu.VMEM((128,), jnp.int32),
        pltpu.VMEM((128, _C), jnp.float32),
        pltpu.VMEM((128, _C), jnp.float32),
        pltpu.VMEM((128, _C), jnp.float32),
        pltpu.VMEM((128, _C), jnp.float32),
        pltpu.VMEM((4, 8, 128), jnp.float32),
        pltpu.SemaphoreType.DMA,
    ],
    compiler_params=_params,
)
def _sample(table_hbm, u5_hbm, v5_hbm, o5_hbm,
            u_v, v_v, fx_v, fy_v,
            i00_v, i01_v, i10_v, i11_v,
            r00_v, r01_v, r10_v, r11_v, o_v, sem):
    cid = lax.axis_index("c")
    sid = lax.axis_index("s")
    wid = sid * 2 + cid
    rows = _H // _NW

    def row_body(rr, carry):
        r = wid * rows + rr
        yt = r // 8
        yi = r - yt * 8

        def chunk_body(xt, c):
            pltpu.sync_copy(u5_hbm.at[yt, xt, yi], u_v)
            pltpu.sync_copy(v5_hbm.at[yt, xt, yi], v_v)

            def idx_grp(g, cc):
                s = g * _L
                uu = u_v[pl.ds(s, _L)]
                vv = v_v[pl.ds(s, _L)]
                x = uu * float(_W) - 0.5
                y = vv * float(_H) - 0.5
                xi = x.astype(jnp.int32)
                yi2 = y.astype(jnp.int32)
                x0 = jnp.where(xi.astype(jnp.float32) > x, xi - 1, xi)
                y0 = jnp.where(yi2.astype(jnp.float32) > y, yi2 - 1, yi2)
                fx_v[pl.ds(s, _L)] = x - x0.astype(jnp.float32)
                fy_v[pl.ds(s, _L)] = y - y0.astype(jnp.float32)
                x0 = jnp.where(x0 < 0, x0 + _W, x0)
                y0 = jnp.where(y0 < 0, y0 + _H, y0)
                x1 = x0 + 1
                x1 = jnp.where(x1 == _W, 0, x1)
                y1 = y0 + 1
                y1 = jnp.where(y1 == _H, 0, y1)
                r0 = y0 * _W
                r1 = y1 * _W
                i00_v[pl.ds(s, _L)] = r0 + x0
                i01_v[pl.ds(s, _L)] = r0 + x1
                i10_v[pl.ds(s, _L)] = r1 + x0
                i11_v[pl.ds(s, _L)] = r1 + x1
                return cc

            lax.fori_loop(0, 128 // _L, idx_grp, 0)

            c00 = pltpu.async_copy(table_hbm.at[i00_v], r00_v, sem)
            c01 = pltpu.async_copy(table_hbm.at[i01_v], r01_v, sem)
            c10 = pltpu.async_copy(table_hbm.at[i10_v], r10_v, sem)
            c11 = pltpu.async_copy(table_hbm.at[i11_v], r11_v, sem)
            c00.wait()
            c01.wait()
            c10.wait()
            c11.wait()

            def blend_grp(g, cc):
                s = g * _L
                fx = fx_v[pl.ds(s, _L)]
                fy = fy_v[pl.ds(s, _L)]
                gx = 1.0 - fx
                gy = 1.0 - fy
                w00 = gx * gy
                w01 = fx * gy
                w10 = gx * fy
                w11 = fx * fy
                pvec = s + lax.iota(jnp.int32, _L)
                for ch in range(_C):
                    cvec = jnp.full((_L,), ch, jnp.int32)
                    g00 = plsc.load_gather(r00_v, [pvec, cvec])
                    g01 = plsc.load_gather(r01_v, [pvec, cvec])
                    g10 = plsc.load_gather(r10_v, [pvec, cvec])
                    g11 = plsc.load_gather(r11_v, [pvec, cvec])
                    o_v[ch // 8, ch % 8, pl.ds(s, _L)] = (
                        g00 * w00 + g01 * w01 + g10 * w10 + g11 * w11
                    )
                return cc

            lax.fori_loop(0, 128 // _L, blend_grp, 0)

            for ct in range(4):
                pltpu.sync_copy(o_v.at[ct], o5_hbm.at[r, ct, xt])
            return c

        lax.fori_loop(0, 8, chunk_body, 0)
        return carry

    lax.fori_loop(0, rows, row_body, 0)


def kernel(texture, u, v):
    tex5 = texture.reshape(_H, 8, 128, 4, 8).transpose(0, 3, 1, 4, 2)
    u5 = u.reshape(128, 8, 8, 128).transpose(0, 2, 1, 3)
    v5 = v.reshape(128, 8, 8, 128).transpose(0, 2, 1, 3)
    table = _convert(tex5)
    o5 = _sample(table, u5, v5)
    return o5.transpose(0, 2, 4, 1, 3).reshape(_H, _W, _C)


# speedup vs baseline: 2.3121x; 2.3121x over previous
import functools

import jax
import jax.numpy as jnp
from jax import lax
from jax.experimental import pallas as pl
from jax.experimental.pallas import tpu as pltpu
from jax.experimental.pallas import tpu_sc as plsc

_L = 16
_H = 1024
_W = 1024
_C = 32
_CP = 33
_NW = 32
_N = _H * _W
_CH = 128

_mesh = plsc.VectorSubcoreMesh(core_axis_name="c", subcore_axis_name="s")
_params = pltpu.CompilerParams(
    use_tc_tiling_on_sc=False, needs_layout_passes=False
)

_BCAST_DNUMS = lax.GatherDimensionNumbers(
    offset_dims=(), collapsed_slice_dims=(0,), start_index_map=(0,)
)


def _lane_bcast(vec, lane):
    sel = jnp.full((_L,), lane, jnp.int32)
    return lax.gather(
        vec,
        sel[:, None],
        _BCAST_DNUMS,
        slice_sizes=(1,),
        mode=lax.GatherScatterMode.PROMISE_IN_BOUNDS,
    )


@functools.partial(
    pl.kernel,
    out_type=jax.ShapeDtypeStruct((_N, _C), jnp.float32),
    mesh=_mesh,
    scratch_types=[
        pltpu.VMEM((4, 8, 8, 128), jnp.float32),
        pltpu.VMEM((_W, _CP), jnp.float32),
    ],
    compiler_params=_params,
)
def _convert(tex5_hbm, table_hbm, in_v, out_v):
    cid = lax.axis_index("c")
    sid = lax.axis_index("s")
    wid = sid * 2 + cid
    planes = _H // _NW

    def plane_body(i, carry):
        y = wid * planes + i
        for ct in range(4):
            pltpu.sync_copy(tex5_hbm.at[y, ct], in_v.at[ct])

        def grp(g, c):
            xt = g // 8
            xg = g - xt * 8
            pvec = g * _L + lax.iota(jnp.int32, _L)
            for ch in range(_C):
                val = in_v[ch // 8, xt, ch % 8, pl.ds(xg * _L, _L)]
                cvec = jnp.full((_L,), ch, jnp.int32)
                plsc.store_scatter(out_v, [pvec, cvec], val)
            return c

        lax.fori_loop(0, _W // _L, grp, 0)
        pltpu.sync_copy(out_v.at[:, pl.ds(0, _C)],
                        table_hbm.at[pl.ds(y * _W, _W)])
        return carry

    lax.fori_loop(0, planes, plane_body, 0)


@functools.partial(
    pl.kernel,
    out_type=jax.ShapeDtypeStruct((_H, 4, 8, 8, 128), jnp.float32),
    mesh=_mesh,
    scratch_types=[
        [pltpu.VMEM((_CH,), jnp.float32) for _ in range(2)],
        [pltpu.VMEM((_CH,), jnp.float32) for _ in range(2)],
        [pltpu.VMEM((_CH,), jnp.float32) for _ in range(2)],
        [pltpu.VMEM((_CH,), jnp.float32) for _ in range(2)],
        [pltpu.VMEM((4, _CH), jnp.int32) for _ in range(2)],
        [pltpu.VMEM((4, _CH, _C), jnp.float32) for _ in range(2)],
        pltpu.VMEM((_CH, _CP), jnp.float32),
        pltpu.VMEM((4, 8, 128), jnp.float32),
        [pltpu.SemaphoreType.DMA for _ in range(2)],
    ],
    compiler_params=_params,
)
def _sample(table_hbm, u5_hbm, v5_hbm, o5_hbm,
            u_v, v_v, fx_v, fy_v, idx_v, rows_v, o_p, o_t, sems):
    cid = lax.axis_index("c")
    sid = lax.axis_index("s")
    wid = sid * 2 + cid
    rows = _H // _NW
    n_chunks = rows * 8

    def stage(k, b):
        r = wid * rows + k // 8
        xt = k - (k // 8) * 8
        yt = r // 8
        yi = r - yt * 8
        pltpu.sync_copy(u5_hbm.at[yt, xt, yi], u_v[b])
        pltpu.sync_copy(v5_hbm.at[yt, xt, yi], v_v[b])

        def idx_grp(g, cc):
            s = g * _L
            uu = u_v[b][pl.ds(s, _L)]
            vv = v_v[b][pl.ds(s, _L)]
            x = uu * float(_W) - 0.5
            y = vv * float(_H) - 0.5
            xi = x.astype(jnp.int32)
            yi2 = y.astype(jnp.int32)
            x0 = jnp.where(xi.astype(jnp.float32) > x, xi - 1, xi)
            y0 = jnp.where(yi2.astype(jnp.float32) > y, yi2 - 1, yi2)
            fx_v[b][pl.ds(s, _L)] = x - x0.astype(jnp.float32)
            fy_v[b][pl.ds(s, _L)] = y - y0.astype(jnp.float32)
            x0 = jnp.where(x0 < 0, x0 + _W, x0)
            y0 = jnp.where(y0 < 0, y0 + _H, y0)
            x1 = x0 + 1
            x1 = jnp.where(x1 == _W, 0, x1)
            y1 = y0 + 1
            y1 = jnp.where(y1 == _H, 0, y1)
            r0 = y0 * _W
            r1 = y1 * _W
            idx_v[b][0, pl.ds(s, _L)] = r0 + x0
            idx_v[b][1, pl.ds(s, _L)] = r0 + x1
            idx_v[b][2, pl.ds(s, _L)] = r1 + x0
            idx_v[b][3, pl.ds(s, _L)] = r1 + x1
            return cc

        lax.fori_loop(0, _CH // _L, idx_grp, 0)
        for c in range(4):
            pltpu.async_copy(table_hbm.at[idx_v[b].at[c]],
                             rows_v[b].at[c], sems[b])

    def finish(k, b):
        r = wid * rows + k // 8
        xt = k - (k // 8) * 8
        for c in range(4):
            pltpu.make_async_copy(table_hbm.at[idx_v[b].at[c]],
                                  rows_v[b].at[c], sems[b]).wait()

        def blend_grp(g, cc):
            s = g * _L
            fx16 = fx_v[b][pl.ds(s, _L)]
            fy16 = fy_v[b][pl.ds(s, _L)]
            for lp in range(_L):
                p = s + lp
                fxp = _lane_bcast(fx16, lp)
                fyp = _lane_bcast(fy16, lp)
                gxp = 1.0 - fxp
                gyp = 1.0 - fyp
                w00 = gxp * gyp
                w01 = fxp * gyp
                w10 = gxp * fyp
                w11 = fxp * fyp
                for half in range(_C // _L):
                    cs = half * _L
                    v00 = rows_v[b][0, p, pl.ds(cs, _L)]
                    v01 = rows_v[b][1, p, pl.ds(cs, _L)]
                    v10 = rows_v[b][2, p, pl.ds(cs, _L)]
                    v11 = rows_v[b][3, p, pl.ds(cs, _L)]
                    o_p[p, pl.ds(cs, _L)] = (
                        v00 * w00 + v01 * w01 + v10 * w10 + v11 * w11
                    )
            return cc

        lax.fori_loop(0, _CH // _L, blend_grp, 0)

        def tr_grp(g, cc):
            s = g * _L
            pvec = s + lax.iota(jnp.int32, _L)
            for ch in range(_C):
                cvec = jnp.full((_L,), ch, jnp.int32)
                o_t[ch // 8, ch % 8, pl.ds(s, _L)] = plsc.load_gather(
                    o_p, [pvec, cvec])
            return cc

        lax.fori_loop(0, _CH // _L, tr_grp, 0)
        for ct in range(4):
            pltpu.sync_copy(o_t.at[ct], o5_hbm.at[r, ct, xt])

    stage(0, 0)

    def pair_body(kk, carry):
        k = kk * 2

        @pl.when(k + 1 < n_chunks)
        def _():
            stage(k + 1, 1)

        finish(k, 0)

        @pl.when(k + 2 < n_chunks)
        def _():
            stage(k + 2, 0)

        @pl.when(k + 1 < n_chunks)
        def _():
            finish(k + 1, 1)

        return carry

    lax.fori_loop(0, (n_chunks + 1) // 2, pair_body, 0)


def kernel(texture, u, v):
    tex5 = texture.reshape(_H, 8, 128, 4, 8).transpose(0, 3, 1, 4, 2)
    u5 = u.reshape(128, 8, 8, 128).transpose(0, 2, 1, 3)
    v5 = v.reshape(128, 8, 8, 128).transpose(0, 2, 1, 3)
    table = _convert(tex5)
    o5 = _sample(table, u5, v5)
    return o5.transpose(0, 2, 4, 1, 3).reshape(_H, _W, _C)


# speedup vs baseline: 3.7247x; 1.6109x over previous
import functools

import jax
import jax.numpy as jnp
from jax import lax
from jax.experimental import pallas as pl
from jax.experimental.pallas import tpu as pltpu
from jax.experimental.pallas import tpu_sc as plsc

_L = 16
_H = 1024
_W = 1024
_C = 32
_CP = 33
_NW = 32
_N = _H * _W
_CH = 128
_HX = 512

_mesh = plsc.VectorSubcoreMesh(core_axis_name="c", subcore_axis_name="s")
_params = pltpu.CompilerParams(
    use_tc_tiling_on_sc=False, needs_layout_passes=False
)

_BCAST_DNUMS = lax.GatherDimensionNumbers(
    offset_dims=(), collapsed_slice_dims=(0,), start_index_map=(0,)
)


def _lane_bcast(vec, lane):
    sel = jnp.full((_L,), lane, jnp.int32)
    return lax.gather(
        vec,
        sel[:, None],
        _BCAST_DNUMS,
        slice_sizes=(1,),
        mode=lax.GatherScatterMode.PROMISE_IN_BOUNDS,
    )


@functools.partial(
    pl.kernel,
    out_type=jax.ShapeDtypeStruct((_N, _C), jnp.float32),
    mesh=_mesh,
    scratch_types=[
        [pltpu.VMEM((4, 4, 8, 128), jnp.float32) for _ in range(2)],
        pltpu.VMEM((_HX, _CP), jnp.float32),
        [pltpu.VMEM((_HX, _C), jnp.float32) for _ in range(2)],
        [pltpu.SemaphoreType.DMA for _ in range(2)],
        [pltpu.SemaphoreType.DMA for _ in range(2)],
    ],
    compiler_params=_params,
)
def _convert(tex5_hbm, table_hbm, in_v, t_v, c_v, isems, osems):
    cid = lax.axis_index("c")
    sid = lax.axis_index("s")
    wid = sid * 2 + cid
    steps = 2 * _H // _NW

    def fire_in(h, b):
        y = wid * (steps // 2) + h // 2
        xh = (h % 2) * 4
        for ct in range(4):
            pltpu.async_copy(tex5_hbm.at[y, ct, pl.ds(xh, 4)],
                             in_v[b].at[ct], isems[b])

    def wait_in(b):
        for ct in range(4):
            pltpu.make_async_copy(tex5_hbm.at[0, 0, pl.ds(0, 4)],
                                  in_v[b].at[ct], isems[b]).wait()

    def process(h, b):
        @plsc.parallel_loop(0, _HX // _L)
        def _grp(g):
            xt = g // 8
            xg = g - xt * 8
            pvec = g * _L + lax.iota(jnp.int32, _L)
            for ch in range(_C):
                val = in_v[b][ch // 8, xt, ch % 8, pl.ds(xg * _L, _L)]
                cvec = jnp.full((_L,), ch, jnp.int32)
                plsc.store_scatter(t_v, [pvec, cvec], val)

        @pl.when(h >= 2)
        def _():
            pltpu.make_async_copy(c_v[b], table_hbm.at[pl.ds(0, _HX)],
                                  osems[b]).wait()

        @plsc.parallel_loop(0, _HX)
        def _cmp(p):
            for half in range(_C // _L):
                c_v[b][p, pl.ds(half * _L, _L)] = t_v[p, pl.ds(half * _L, _L)]

        y = wid * (steps // 2) + h // 2
        pltpu.async_copy(
            c_v[b], table_hbm.at[pl.ds(y * _W + (h % 2) * _HX, _HX)],
            osems[b])

    fire_in(0, 0)

    def pair_body(hh, carry):
        h = hh * 2
        wait_in(0)
        fire_in(h + 1, 1)
        process(h, 0)
        wait_in(1)

        @pl.when(h + 2 < steps)
        def _():
            fire_in(h + 2, 0)

        process(h + 1, 1)
        return carry

    lax.fori_loop(0, steps // 2, pair_body, 0)
    for b in range(2):
        pltpu.make_async_copy(c_v[b], table_hbm.at[pl.ds(0, _HX)],
                              osems[b]).wait()


@functools.partial(
    pl.kernel,
    out_type=jax.ShapeDtypeStruct((_H, 4, 8, 8, 128), jnp.float32),
    mesh=_mesh,
    scratch_types=[
        [pltpu.VMEM((_CH,), jnp.float32) for _ in range(2)],
        [pltpu.VMEM((_CH,), jnp.float32) for _ in range(2)],
        [pltpu.VMEM((_CH,), jnp.float32) for _ in range(2)],
        [pltpu.VMEM((_CH,), jnp.float32) for _ in range(2)],
        [pltpu.VMEM((4, _CH), jnp.int32) for _ in range(2)],
        [pltpu.VMEM((4, _CH, _C), jnp.float32) for _ in range(2)],
        pltpu.VMEM((_CH, _CP), jnp.float32),
        [pltpu.VMEM((4, 8, 128), jnp.float32) for _ in range(2)],
        [pltpu.SemaphoreType.DMA for _ in range(2)],
        [pltpu.SemaphoreType.DMA for _ in range(2)],
    ],
    compiler_params=_params,
)
def _sample(table_hbm, u5_hbm, v5_hbm, o5_hbm,
            u_v, v_v, fx_v, fy_v, idx_v, rows_v, o_p, o_t, gsems, osems):
    cid = lax.axis_index("c")
    sid = lax.axis_index("s")
    wid = sid * 2 + cid
    rows = _H // _NW
    n_chunks = rows * 8

    def stage(k, b):
        r = wid * rows + k // 8
        xt = k - (k // 8) * 8
        yt = r // 8
        yi = r - yt * 8
        pltpu.sync_copy(u5_hbm.at[yt, xt, yi], u_v[b])
        pltpu.sync_copy(v5_hbm.at[yt, xt, yi], v_v[b])

        @plsc.parallel_loop(0, _CH // _L)
        def _idx(g):
            s = g * _L
            uu = u_v[b][pl.ds(s, _L)]
            vv = v_v[b][pl.ds(s, _L)]
            x = uu * float(_W) - 0.5
            y = vv * float(_H) - 0.5
            xi = x.astype(jnp.int32)
            yi2 = y.astype(jnp.int32)
            x0 = jnp.where(xi.astype(jnp.float32) > x, xi - 1, xi)
            y0 = jnp.where(yi2.astype(jnp.float32) > y, yi2 - 1, yi2)
            fx_v[b][pl.ds(s, _L)] = x - x0.astype(jnp.float32)
            fy_v[b][pl.ds(s, _L)] = y - y0.astype(jnp.float32)
            x0 = jnp.where(x0 < 0, x0 + _W, x0)
            y0 = jnp.where(y0 < 0, y0 + _H, y0)
            x1 = x0 + 1
            x1 = jnp.where(x1 == _W, 0, x1)
            y1 = y0 + 1
            y1 = jnp.where(y1 == _H, 0, y1)
            r0 = y0 * _W
            r1 = y1 * _W
            idx_v[b][0, pl.ds(s, _L)] = r0 + x0
            idx_v[b][1, pl.ds(s, _L)] = r0 + x1
            idx_v[b][2, pl.ds(s, _L)] = r1 + x0
            idx_v[b][3, pl.ds(s, _L)] = r1 + x1

        for c in range(4):
            pltpu.async_copy(table_hbm.at[idx_v[b].at[c]],
                             rows_v[b].at[c], gsems[b])

    def finish(k, b, first):
        r = wid * rows + k // 8
        xt = k - (k // 8) * 8
        for c in range(4):
            pltpu.make_async_copy(table_hbm.at[idx_v[b].at[c]],
                                  rows_v[b].at[c], gsems[b]).wait()

        @plsc.parallel_loop(0, _CH // _L)
        def _blend(g):
            s = g * _L
            fx16 = fx_v[b][pl.ds(s, _L)]
            fy16 = fy_v[b][pl.ds(s, _L)]
            for lp in range(_L):
                p = s + lp
                fxp = _lane_bcast(fx16, lp)
                fyp = _lane_bcast(fy16, lp)
                gxp = 1.0 - fxp
                gyp = 1.0 - fyp
                w00 = gxp * gyp
                w01 = fxp * gyp
                w10 = gxp * fyp
                w11 = fxp * fyp
                for half in range(_C // _L):
                    cs = half * _L
                    v00 = rows_v[b][0, p, pl.ds(cs, _L)]
                    v01 = rows_v[b][1, p, pl.ds(cs, _L)]
                    v10 = rows_v[b][2, p, pl.ds(cs, _L)]
                    v11 = rows_v[b][3, p, pl.ds(cs, _L)]
                    o_p[p, pl.ds(cs, _L)] = (
                        v00 * w00 + v01 * w01 + v10 * w10 + v11 * w11
                    )

        @pl.when(jnp.logical_not(first))
        def _():
            for ct in range(4):
                pltpu.make_async_copy(o_t[b].at[ct],
                                      o5_hbm.at[0, ct, 0], osems[b]).wait()

        @plsc.parallel_loop(0, _CH // _L)
        def _tr(g):
            s = g * _L
            pvec = s + lax.iota(jnp.int32, _L)
            for ch in range(_C):
                cvec = jnp.full((_L,), ch, jnp.int32)
                o_t[b][ch // 8, ch % 8, pl.ds(s, _L)] = plsc.load_gather(
                    o_p, [pvec, cvec])

        for ct in range(4):
            pltpu.async_copy(o_t[b].at[ct], o5_hbm.at[r, ct, xt], osems[b])

    stage(0, 0)

    def pair_body(kk, carry):
        k = kk * 2
        stage(k + 1, 1)
        finish(k, 0, kk == 0)

        @pl.when(k + 2 < n_chunks)
        def _():
            stage(k + 2, 0)

        finish(k + 1, 1, kk == 0)
        return carry

    lax.fori_loop(0, n_chunks // 2, pair_body, 0)
    for b in range(2):
        for ct in range(4):
            pltpu.make_async_copy(o_t[b].at[ct], o5_hbm.at[0, ct, 0],
                                  osems[b]).wait()


def kernel(texture, u, v):
    tex5 = texture.reshape(_H, 8, 128, 4, 8).transpose(0, 3, 1, 4, 2)
    u5 = u.reshape(128, 8, 8, 128).transpose(0, 2, 1, 3)
    v5 = v.reshape(128, 8, 8, 128).transpose(0, 2, 1, 3)
    table = _convert(tex5)
    o5 = _sample(table, u5, v5)
    return o5.transpose(0, 2, 4, 1, 3).reshape(_H, _W, _C)
